# baseline (speedup 1.0000x reference)
import functools

import numpy as np

import jax
import jax.numpy as jnp
from jax import lax
from jax.experimental import pallas as pl
from jax.experimental.pallas import tpu as pltpu
from jax.experimental.pallas import tpu_sc as plsc

N_NODES = 10000
N_EDGES = 320000
D = 128

NC = 2
NS = 16
NW = NC * NS

NPAD = 10240
EPAD = 327680
EPT = EPAD // NW
CH = 128
NCHUNK = EPT // CH
NB = 2
NI = 4
RPT = NPAD // NS
RZ = 128
WDEG = 16


def _zero_acc(zero_hbm, acc, sid):
  for r in range(RPT // RZ):
    pltpu.sync_copy(zero_hbm, acc.at[pl.ds(sid * RPT + r * RZ, RZ)])


def _read_acc(acc, out_hbm, cid, sid):
  for r in range(RPT // RZ):
    rows = pl.ds(sid * RPT + r * RZ, RZ)
    pltpu.sync_copy(acc.at[rows], out_hbm.at[cid, rows])


def _sc_degree_body(padj_hbm, out_hbm, stage, hist_v, didx_v, red_v, bcast_v):
  cid = lax.axis_index("c")
  sid = lax.axis_index("s")
  wid = cid * NS + sid

  @pl.loop(0, NPAD // 16)
  def _(k):
    hist_v[pl.ds(k * 16, 16)] = jnp.zeros((16,), jnp.float32)

  ept = N_EDGES // NW
  pltpu.sync_copy(padj_hbm.at[pl.ds(wid * ept, ept)], didx_v)
  ones16 = jnp.ones((16,), jnp.float32)

  @pl.loop(0, ept // 16)
  def _(j):
    plsc.addupdate_scatter(hist_v, [didx_v[pl.ds(j * 16, 16)]], ones16)

  pltpu.sync_copy(hist_v, stage.at[sid])
  plsc.subcore_barrier()
  for t in range(NS):
    pltpu.sync_copy(stage.at[t, pl.ds(sid * RPT, RPT)], red_v.at[t])

  @pl.loop(0, RPT // 16)
  def _(k):
    s = red_v[0, pl.ds(k * 16, 16)]
    for t in range(1, NS):
      s = s + red_v[t, pl.ds(k * 16, 16)]
    hist_v[pl.ds(k * 16, 16)] = s

  zero16 = lax.iota(jnp.int32, 16) * 0

  @pl.loop(0, RPT)
  def _(m):
    bcast_v[m, :] = plsc.load_gather(hist_v, [m + zero16])

  pltpu.sync_copy(bcast_v, out_hbm.at[cid, pl.ds(sid * RPT, RPT)])


def _sc_prop_body(h_hbm, padj_hbm, zero_hbm, out_hbm, acc, pidx_v, sidx_v,
                  didx_v, rows_v, *sems):
  gsems, ssems, isems = sems[:NB], sems[NB:2 * NB], sems[2 * NB:]
  cid = lax.axis_index("c")
  sid = lax.axis_index("s")
  wid = cid * NS + sid
  def idx_start(j, ib):
    pltpu.async_copy(padj_hbm.at[pl.ds(wid * EPT + j * CH, CH)],
                     pidx_v.at[ib], isems[ib])

  def idx_wait(ib):
    pltpu.make_async_copy(padj_hbm.at[pl.ds(0, CH)],
                          pidx_v.at[ib], isems[ib]).wait()
    for k in range(CH // 16):
      v = pidx_v[ib, pl.ds(k * 16, 16)]
      sidx_v[ib, pl.ds(k * 16, 16)] = lax.shift_right_logical(v, 16)
      didx_v[ib, pl.ds(k * 16, 16)] = v & 0xFFFF

  def gather_start(ib, b):
    pltpu.async_copy(h_hbm.at[sidx_v.at[ib]], rows_v.at[b], gsems[b])

  def gather_wait(b):
    pltpu.make_async_copy(h_hbm.at[sidx_v.at[0]], rows_v.at[b],
                          gsems[b]).wait()

  def scatter_start(ib, b):
    pltpu.async_copy(rows_v.at[b], acc.at[didx_v.at[ib]], ssems[b],
                     add=True)

  def scatter_wait(b):
    pltpu.make_async_copy(rows_v.at[b], acc.at[pl.ds(0, CH)],
                          ssems[b]).wait()

  def body(j):
    b0, b1 = j % NB, (j + 1) % NB
    if j >= 1:
      scatter_wait(b1)
    if j + 3 < NCHUNK:
      idx_start(j + 3, (j + 3) % NI)
    if j + 1 < NCHUNK:
      idx_wait((j + 1) % NI)
      gather_start((j + 1) % NI, b1)
    gather_wait(b0)
    scatter_start(j % NI, b0)

  for j in range(3):
    idx_start(j, j)
  idx_wait(0)
  gather_start(0, 0)
  _zero_acc(zero_hbm, acc, sid)
  plsc.subcore_barrier()
  body(0)

  steady = (NCHUNK - 4) // 4

  @pl.loop(0, steady)
  def _(jo):
    for u in range(4):
      j = 1 + jo * 4 + u
      scatter_wait((u + 2) % NB)
      idx_start(j + 3, u % NI)
      idx_wait((u + 2) % NI)
      gather_start((u + 2) % NI, (u + 2) % NB)
      gather_wait((u + 1) % NB)
      scatter_start((u + 1) % NI, (u + 1) % NB)

  for j in range(1 + steady * 4, NCHUNK):
    body(j)
  scatter_wait((NCHUNK - 1) % NB)

  plsc.subcore_barrier()
  _read_acc(acc, out_hbm, cid, sid)


@functools.cache
def _sc_kernels():
  mesh = plsc.VectorSubcoreMesh(
      core_axis_name="c", subcore_axis_name="s",
      num_cores=NC, num_subcores=NS)
  sc_degree = pl.kernel(
      _sc_degree_body,
      out_type=jax.ShapeDtypeStruct((NC, NPAD, WDEG), jnp.float32),
      mesh=mesh,
      compiler_params=pltpu.CompilerParams(needs_layout_passes=False),
      scratch_types=[
          pltpu.VMEM_SHARED((NS, NPAD), jnp.float32),
          pltpu.VMEM((NPAD,), jnp.float32),
          pltpu.VMEM((N_EDGES // NW,), jnp.int32),
          pltpu.VMEM((NS, RPT), jnp.float32),
          pltpu.VMEM((RPT, WDEG), jnp.float32),
      ])
  sc_prop = pl.kernel(
      _sc_prop_body,
      out_type=jax.ShapeDtypeStruct((NC, NPAD, D), jnp.float32),
      mesh=mesh,
      scratch_types=[
          pltpu.VMEM_SHARED((NPAD, D), jnp.float32),
          pltpu.VMEM((NI, CH), jnp.int32),
          pltpu.VMEM((NI, CH), jnp.int32),
          pltpu.VMEM((NI, CH), jnp.int32),
          pltpu.VMEM((NB, CH, D), jnp.float32),
      ] + [pltpu.SemaphoreType.DMA] * (2 * NB + NI))
  return sc_degree, sc_prop


BLK = 2048
_GRID = NPAD // BLK


def _dis_of(deg_blk):
  return jnp.where(deg_blk > 0.0,
                   lax.rsqrt(jnp.maximum(deg_blk, 1e-12)), 0.0)


def _tc_first_body(x_ref, w_ref, degp_ref, h_ref, dis_ref):
  deg = degp_ref[0] + degp_ref[1]
  dis = _dis_of(deg)
  h = jnp.dot(x_ref[...], w_ref[...], preferred_element_type=jnp.float32)
  h_ref[...] = dis[:, 0:1] * h
  dis_ref[...] = dis


def _tc_first(x, w, deg_parts):
  return pl.pallas_call(
      _tc_first_body,
      grid=(_GRID,),
      in_specs=[
          pl.BlockSpec((BLK, D), lambda i: (i, 0)),
          pl.BlockSpec((D, D), lambda i: (0, 0)),
          pl.BlockSpec((NC, BLK, WDEG), lambda i: (0, i, 0)),
      ],
      out_specs=[
          pl.BlockSpec((BLK, D), lambda i: (i, 0)),
          pl.BlockSpec((BLK, WDEG), lambda i: (i, 0)),
      ],
      out_shape=[
          jax.ShapeDtypeStruct((NPAD, D), jnp.float32),
          jax.ShapeDtypeStruct((NPAD, WDEG), jnp.float32),
      ],
  )(x, w, deg_parts)


def _tc_mid_body(p_ref, dis_ref, b_ref, w_ref, h_ref):
  d = dis_ref[:, 0:1]
  y = jnp.maximum(d * (p_ref[0] + p_ref[1]) + b_ref[...], 0.0)
  h_ref[...] = d * jnp.dot(y, w_ref[...], preferred_element_type=jnp.float32)


def _tc_mid(parts, dis, b, w):
  return pl.pallas_call(
      _tc_mid_body,
      grid=(_GRID,),
      in_specs=[
          pl.BlockSpec((NC, BLK, D), lambda i: (0, i, 0)),
          pl.BlockSpec((BLK, WDEG), lambda i: (i, 0)),
          pl.BlockSpec((1, D), lambda i: (0, 0)),
          pl.BlockSpec((D, D), lambda i: (0, 0)),
      ],
      out_specs=pl.BlockSpec((BLK, D), lambda i: (i, 0)),
      out_shape=jax.ShapeDtypeStruct((NPAD, D), jnp.float32),
  )(parts, dis, b, w)


def _tc_last_body(p_ref, dis_ref, b_ref, o_ref):
  o_ref[...] = dis_ref[:, 0:1] * (p_ref[0] + p_ref[1]) + b_ref[...]


BLKL = 2000


def _tc_last(parts, dis, b):
  return pl.pallas_call(
      _tc_last_body,
      grid=(N_NODES // BLKL,),
      in_specs=[
          pl.BlockSpec((NC, BLKL, D), lambda i: (0, i, 0)),
          pl.BlockSpec((BLKL, WDEG), lambda i: (i, 0)),
          pl.BlockSpec((1, D), lambda i: (0, 0)),
      ],
      out_specs=pl.BlockSpec((BLKL, D), lambda i: (i, 0)),
      out_shape=jax.ShapeDtypeStruct((N_NODES, D), jnp.float32),
  )(parts, dis, b)


def kernel(x, adj_t, W1, b1, W2, b2, W3, b3):
  pad_i = N_NODES + np.arange(EPAD - N_EDGES, dtype=np.int32) % (
      NPAD - N_NODES)
  pad_packed = jnp.asarray((pad_i << 16) | pad_i)
  packed = jnp.concatenate([(adj_t[0] << 16) | adj_t[1], pad_packed])
  x_p = jnp.pad(x, ((0, NPAD - N_NODES), (0, 0)))
  zero_rows = jnp.zeros((RZ, D), jnp.float32)
  b1r, b2r, b3r = (b.reshape(1, D) for b in (b1, b2, b3))

  _sc_degree, _sc_prop = _sc_kernels()
  deg_parts = _sc_degree(adj_t[1])
  h, dis = _tc_first(x_p, W1, deg_parts)
  p = _sc_prop(h, packed, zero_rows)
  h = _tc_mid(p, dis, b1r, W2)
  p = _sc_prop(h, packed, zero_rows)
  h = _tc_mid(p, dis, b2r, W3)
  p = _sc_prop(h, packed, zero_rows)
  return _tc_last(p, dis, b3r)


# speedup vs baseline: 1.0367x; 1.0367x over previous
import functools

import numpy as np

import jax
import jax.numpy as jnp
from jax import lax
from jax.experimental import pallas as pl
from jax.experimental.pallas import tpu as pltpu
from jax.experimental.pallas import tpu_sc as plsc

N_NODES = 10000
N_EDGES = 320000
D = 128

NC = 2
NS = 16
NW = NC * NS

NPAD = 10240
EPAD = 327680
EPT = EPAD // NW
CH = 128
NCHUNK = EPT // CH
NB = 2
NI = 4
RPT = NPAD // NS
RZ = 128
WDEG = 16


def _zero_acc(zero_hbm, acc, sid):
  for r in range(RPT // RZ):
    pltpu.sync_copy(zero_hbm, acc.at[pl.ds(sid * RPT + r * RZ, RZ)])


def _read_acc(acc, out_hbm, cid, sid):
  for r in range(RPT // RZ):
    rows = pl.ds(sid * RPT + r * RZ, RZ)
    pltpu.sync_copy(acc.at[rows], out_hbm.at[cid, rows])


def _sc_degree_body(padj_hbm, out_hbm, stage, hist_v, didx_v, red_v, bcast_v):
  cid = lax.axis_index("c")
  sid = lax.axis_index("s")
  wid = cid * NS + sid

  @pl.loop(0, NPAD // 16)
  def _(k):
    hist_v[pl.ds(k * 16, 16)] = jnp.zeros((16,), jnp.float32)

  ept = N_EDGES // NW
  pltpu.sync_copy(padj_hbm.at[pl.ds(wid * ept, ept)], didx_v)
  ones16 = jnp.ones((16,), jnp.float32)

  @pl.loop(0, ept // 16)
  def _(j):
    plsc.addupdate_scatter(hist_v, [didx_v[pl.ds(j * 16, 16)]], ones16)

  pltpu.sync_copy(hist_v, stage.at[sid])
  plsc.subcore_barrier()
  for t in range(NS):
    pltpu.sync_copy(stage.at[t, pl.ds(sid * RPT, RPT)], red_v.at[t])

  @pl.loop(0, RPT // 16)
  def _(k):
    s = red_v[0, pl.ds(k * 16, 16)]
    for t in range(1, NS):
      s = s + red_v[t, pl.ds(k * 16, 16)]
    hist_v[pl.ds(k * 16, 16)] = s

  zero16 = lax.iota(jnp.int32, 16) * 0

  @pl.loop(0, RPT)
  def _(m):
    bcast_v[m, :] = plsc.load_gather(hist_v, [m + zero16])

  pltpu.sync_copy(bcast_v, out_hbm.at[cid, pl.ds(sid * RPT, RPT)])


def _sc_prop_body(h_hbm, padj_hbm, zero_hbm, out_hbm, acc, pidx_v, sidx_v,
                  didx_v, rows_v, *sems):
  gsems, ssems, isems = sems[:NB], sems[NB:2 * NB], sems[2 * NB:]
  cid = lax.axis_index("c")
  sid = lax.axis_index("s")
  wid = cid * NS + sid
  def idx_start(j, ib):
    pltpu.async_copy(padj_hbm.at[pl.ds(wid * NCHUNK + j, 1)],
                     pidx_v.at[pl.ds(ib, 1)], isems[ib])

  def idx_wait(ib):
    pltpu.make_async_copy(padj_hbm.at[pl.ds(0, 1)],
                          pidx_v.at[pl.ds(ib, 1)], isems[ib]).wait()
    for k in range(CH // 16):
      v = pidx_v[ib, pl.ds(k * 16, 16)]
      sidx_v[ib, pl.ds(k * 16, 16)] = lax.shift_right_logical(v, 16)
      didx_v[ib, pl.ds(k * 16, 16)] = v & 0xFFFF

  def gather_start(ib, b):
    pltpu.async_copy(h_hbm.at[sidx_v.at[ib]], rows_v.at[b], gsems[b])

  def gather_wait(b):
    pltpu.make_async_copy(h_hbm.at[sidx_v.at[0]], rows_v.at[b],
                          gsems[b]).wait()

  def scatter_start(ib, b):
    pltpu.async_copy(rows_v.at[b], acc.at[didx_v.at[ib]], ssems[b],
                     add=True)

  def scatter_wait(b):
    pltpu.make_async_copy(rows_v.at[b], acc.at[pl.ds(0, CH)],
                          ssems[b]).wait()

  def body(j):
    b0, b1 = j % NB, (j + 1) % NB
    if j >= 1:
      scatter_wait(b1)
    if j + 3 < NCHUNK:
      idx_start(j + 3, (j + 3) % NI)
    if j + 1 < NCHUNK:
      idx_wait((j + 1) % NI)
      gather_start((j + 1) % NI, b1)
    gather_wait(b0)
    scatter_start(j % NI, b0)

  for j in range(3):
    idx_start(j, j)
  idx_wait(0)
  gather_start(0, 0)
  _zero_acc(zero_hbm, acc, sid)
  plsc.subcore_barrier()
  body(0)

  steady = (NCHUNK - 4) // 4

  @pl.loop(0, steady)
  def _(jo):
    for u in range(4):
      j = 1 + jo * 4 + u
      scatter_wait((u + 2) % NB)
      idx_start(j + 3, u % NI)
      idx_wait((u + 2) % NI)
      gather_start((u + 2) % NI, (u + 2) % NB)
      gather_wait((u + 1) % NB)
      scatter_start((u + 1) % NI, (u + 1) % NB)

  for j in range(1 + steady * 4, NCHUNK):
    body(j)
  scatter_wait((NCHUNK - 1) % NB)

  plsc.subcore_barrier()
  _read_acc(acc, out_hbm, cid, sid)


@functools.cache
def _sc_kernels():
  mesh = plsc.VectorSubcoreMesh(
      core_axis_name="c", subcore_axis_name="s",
      num_cores=NC, num_subcores=NS)
  sc_degree = pl.kernel(
      _sc_degree_body,
      out_type=jax.ShapeDtypeStruct((NC, NPAD, WDEG), jnp.float32),
      mesh=mesh,
      compiler_params=pltpu.CompilerParams(needs_layout_passes=False),
      scratch_types=[
          pltpu.VMEM_SHARED((NS, NPAD), jnp.float32),
          pltpu.VMEM((NPAD,), jnp.float32),
          pltpu.VMEM((N_EDGES // NW,), jnp.int32),
          pltpu.VMEM((NS, RPT), jnp.float32),
          pltpu.VMEM((RPT, WDEG), jnp.float32),
      ])
  sc_prop = pl.kernel(
      _sc_prop_body,
      out_type=jax.ShapeDtypeStruct((NC, NPAD, D), jnp.float32),
      mesh=mesh,
      scratch_types=[
          pltpu.VMEM_SHARED((NPAD, D), jnp.float32),
          pltpu.VMEM((NI, CH), jnp.int32),
          pltpu.VMEM((NI, CH), jnp.int32),
          pltpu.VMEM((NI, CH), jnp.int32),
          pltpu.VMEM((NB, CH, D), jnp.float32),
      ] + [pltpu.SemaphoreType.DMA] * (2 * NB + NI))
  return sc_degree, sc_prop


BLK = NPAD
_GRID = NPAD // BLK


def _dis_of(deg_blk):
  return jnp.where(deg_blk > 0.0,
                   lax.rsqrt(jnp.maximum(deg_blk, 1e-12)), 0.0)


def _tc_first_body(x_ref, w_ref, degp_ref, h_ref, dis_ref):
  deg = degp_ref[0] + degp_ref[1]
  dis = _dis_of(deg)
  h = jnp.dot(x_ref[...], w_ref[...], preferred_element_type=jnp.float32)
  h_ref[...] = dis[:, 0:1] * h
  dis_ref[...] = dis


def _tc_first(x, w, deg_parts):
  return pl.pallas_call(
      _tc_first_body,
      grid=(_GRID,),
      in_specs=[
          pl.BlockSpec((BLK, D), lambda i: (i, 0)),
          pl.BlockSpec((D, D), lambda i: (0, 0)),
          pl.BlockSpec((NC, BLK, WDEG), lambda i: (0, i, 0)),
      ],
      out_specs=[
          pl.BlockSpec((BLK, D), lambda i: (i, 0)),
          pl.BlockSpec((BLK, WDEG), lambda i: (i, 0)),
      ],
      out_shape=[
          jax.ShapeDtypeStruct((NPAD, D), jnp.float32),
          jax.ShapeDtypeStruct((NPAD, WDEG), jnp.float32),
      ],
  )(x, w, deg_parts)


def _tc_mid_body(p_ref, dis_ref, b_ref, w_ref, h_ref):
  d = dis_ref[:, 0:1]
  y = jnp.maximum(d * (p_ref[0] + p_ref[1]) + b_ref[...], 0.0)
  h_ref[...] = d * jnp.dot(y, w_ref[...], preferred_element_type=jnp.float32)


def _tc_mid(parts, dis, b, w):
  return pl.pallas_call(
      _tc_mid_body,
      grid=(_GRID,),
      in_specs=[
          pl.BlockSpec((NC, BLK, D), lambda i: (0, i, 0)),
          pl.BlockSpec((BLK, WDEG), lambda i: (i, 0)),
          pl.BlockSpec((1, D), lambda i: (0, 0)),
          pl.BlockSpec((D, D), lambda i: (0, 0)),
      ],
      out_specs=pl.BlockSpec((BLK, D), lambda i: (i, 0)),
      out_shape=jax.ShapeDtypeStruct((NPAD, D), jnp.float32),
  )(parts, dis, b, w)


def _tc_last_body(p_ref, dis_ref, b_ref, o_ref):
  o_ref[...] = dis_ref[:, 0:1] * (p_ref[0] + p_ref[1]) + b_ref[...]


BLKL = N_NODES


def _tc_last(parts, dis, b):
  return pl.pallas_call(
      _tc_last_body,
      grid=(N_NODES // BLKL,),
      in_specs=[
          pl.BlockSpec((NC, BLKL, D), lambda i: (0, i, 0)),
          pl.BlockSpec((BLKL, WDEG), lambda i: (i, 0)),
          pl.BlockSpec((1, D), lambda i: (0, 0)),
      ],
      out_specs=pl.BlockSpec((BLKL, D), lambda i: (i, 0)),
      out_shape=jax.ShapeDtypeStruct((N_NODES, D), jnp.float32),
  )(parts, dis, b)


def kernel(x, adj_t, W1, b1, W2, b2, W3, b3):
  pad_i = N_NODES + np.arange(EPAD - N_EDGES, dtype=np.int32) % (
      NPAD - N_NODES)
  pad_packed = jnp.asarray(((pad_i << 16) | pad_i).reshape(-1, CH))
  packed = jnp.concatenate(
      [(adj_t[0].reshape(-1, CH) << 16) | adj_t[1].reshape(-1, CH),
       pad_packed])
  x_p = jnp.pad(x, ((0, NPAD - N_NODES), (0, 0)))
  zero_rows = jnp.zeros((RZ, D), jnp.float32)
  b1r, b2r, b3r = (b.reshape(1, D) for b in (b1, b2, b3))

  _sc_degree, _sc_prop = _sc_kernels()
  deg_parts = _sc_degree(adj_t[1])
  h, dis = _tc_first(x_p, W1, deg_parts)
  p = _sc_prop(h, packed, zero_rows)
  h = _tc_mid(p, dis, b1r, W2)
  p = _sc_prop(h, packed, zero_rows)
  h = _tc_mid(p, dis, b2r, W3)
  p = _sc_prop(h, packed, zero_rows)
  return _tc_last(p, dis, b3r)
